# baseline (speedup 1.0000x reference)
import jax
import jax.numpy as jnp
from jax import lax
from jax.experimental import pallas as pl
from jax.experimental.pallas import tpu as pltpu
from jax.experimental.pallas import tpu_sc as plsc

NUM_NODES = 1000000
DIM = 64
BATCH = 16384

NC = 2
NS = 16
NW = NC * NS
BPW = BATCH // NW
NI = 128
NCHUNK = BPW // NI
NBLK = NUM_NODES // 4
PKW = 2 * DIM


def _sc_body(u_hbm, v_hbm, epk, cpk, out_hbm,
             idx_u, idx_v, blk_u, blk_v, ebuf, cbuf, out_v, sem_e, sem_c):
    wid = lax.axis_index("s") * NC + lax.axis_index("c")
    pltpu.sync_copy(u_hbm.at[wid], idx_u)
    pltpu.sync_copy(v_hbm.at[wid], idx_v)

    def blkids(g, carry):
        blk_u[pl.ds(g * 16, 16)] = idx_u[pl.ds(g * 16, 16)] >> 2
        blk_v[pl.ds(g * 16, 16)] = idx_v[pl.ds(g * 16, 16)] >> 2
        return carry

    lax.fori_loop(0, BPW // 16, blkids, 0)

    def fire(c, s):
        pltpu.async_copy(
            epk.at[blk_u.at[pl.ds(c * NI, NI)]], ebuf.at[s], sem_e.at[s])
        pltpu.async_copy(
            cpk.at[blk_v.at[pl.ds(c * NI, NI)]], cbuf.at[s], sem_c.at[s])

    def drain(s):
        pltpu.make_async_copy(
            epk.at[blk_u.at[pl.ds(0, NI)]], ebuf.at[s], sem_e.at[s]).wait()
        pltpu.make_async_copy(
            cpk.at[blk_v.at[pl.ds(0, NI)]], cbuf.at[s], sem_c.at[s]).wait()

    iota16 = lax.iota(jnp.int32, 16)
    perms = [iota16 ^ (1 << b) for b in range(4)]
    dnums = lax.GatherDimensionNumbers(
        offset_dims=(), collapsed_slice_dims=(0,), start_index_map=(0,))

    def permute(x, pm):
        return lax.gather(
            x, pm[:, None], dimension_numbers=dnums, slice_sizes=(1,),
            mode=lax.GatherScatterMode.PROMISE_IN_BOUNDS)

    def lanesum(p):
        for pm in perms:
            p = p + permute(p, pm)
        return p

    himask = jnp.full((16,), -65536, jnp.int32)

    def rowhalves(buf, k, off):
        w0 = plsc.bitcast(buf[k, pl.ds(off, 16)], jnp.int32)
        w1 = plsc.bitcast(buf[k, pl.ds(off + 16, 16)], jnp.int32)
        a0 = plsc.bitcast(w0 << 16, jnp.float32)
        b0 = plsc.bitcast(w0 & himask, jnp.float32)
        a1 = plsc.bitcast(w1 << 16, jnp.float32)
        b1 = plsc.bitcast(w1 & himask, jnp.float32)
        return a0, b0, a1, b1

    def compute(c, s):
        def body(g, carry):
            su = (idx_u[pl.ds(c * NI + g * 16, 16)] & 3) * 32
            sv = (idx_v[pl.ds(c * NI + g * 16, 16)] & 3) * 32
            acc = jnp.zeros((16,), jnp.float32)
            for k in range(16):
                ea0, eb0, ea1, eb1 = rowhalves(ebuf.at[s], g * 16 + k, su[k])
                ca0, cb0, ca1, cb1 = rowhalves(cbuf.at[s], g * 16 + k, sv[k])
                p = ea0 * ca0 + eb0 * cb0 + ea1 * ca1 + eb1 * cb1
                acc = jnp.where(iota16 == k, lanesum(p), acc)
            out_v[pl.ds(c * NI + g * 16, 16)] = acc
            return carry

        lax.fori_loop(0, NI // 16, body, 0)

    fire(0, 0)
    fire(1, 1)
    for c in range(NCHUNK):
        s = c % 2
        drain(s)
        compute(c, s)
        if c + 2 < NCHUNK:
            fire(c + 2, s)

    pltpu.sync_copy(out_v, out_hbm.at[pl.ds(wid * BPW, BPW)])


_sc_dot = pl.kernel(
    _sc_body,
    out_type=jax.ShapeDtypeStruct((BATCH,), jnp.float32),
    mesh=plsc.VectorSubcoreMesh(core_axis_name="c", subcore_axis_name="s"),
    compiler_params=pltpu.CompilerParams(
        use_tc_tiling_on_sc=False, needs_layout_passes=False),
    scratch_types=[
        pltpu.VMEM((BPW,), jnp.int32),
        pltpu.VMEM((BPW,), jnp.int32),
        pltpu.VMEM((BPW,), jnp.int32),
        pltpu.VMEM((BPW,), jnp.int32),
        pltpu.VMEM((2, NI, PKW), jnp.float32),
        pltpu.VMEM((2, NI, PKW), jnp.float32),
        pltpu.VMEM((BPW,), jnp.float32),
        pltpu.SemaphoreType.DMA((2,)),
        pltpu.SemaphoreType.DMA((2,)),
    ],
)


def _loss_body(p_ref, s_ref, o_ref):
    x = s_ref[...] * p_ref[...]
    ls = jnp.minimum(x, 0.0) - jnp.log1p(jnp.exp(-jnp.abs(x)))
    o_ref[0, 0] = -jnp.sum(ls)


_loss = pl.pallas_call(
    _loss_body,
    out_shape=jax.ShapeDtypeStruct((1, 1), jnp.float32),
    out_specs=pl.BlockSpec(memory_space=pltpu.SMEM),
)


def _pack(table):
    bf = table.astype(jnp.bfloat16).reshape(NBLK, PKW, 2)
    return jax.lax.bitcast_convert_type(bf, jnp.float32)


def kernel(u, v, sign, emb_table, ctx_table):
    u2 = u.astype(jnp.int32).reshape(NW, BPW)
    v2 = v.astype(jnp.int32).reshape(NW, BPW)
    prod = _sc_dot(u2, v2, _pack(emb_table), _pack(ctx_table))
    loss = _loss(prod.reshape(128, 128), sign.reshape(128, 128))
    return loss.reshape(())


# speedup vs baseline: 46.6690x; 46.6690x over previous
import jax
import jax.numpy as jnp
from jax import lax
from jax.experimental import pallas as pl
from jax.experimental.pallas import tpu as pltpu
from jax.experimental.pallas import tpu_sc as plsc

NUM_NODES = 1000000
DIM = 64
BATCH = 16384

NC = 2
NS = 16
NW = NC * NS
BPW = BATCH // NW
NI = 128
NCHUNK = BPW // NI
PKW = 2 * DIM
BN = 2048
NSTEP = pl.cdiv(NUM_NODES, BN)
NBLK = NSTEP * (BN // 4)


def _pack_body(t_ref, o_ref):
    x = lax.bitcast_convert_type(t_ref[...], jnp.int32)
    r = (x + 0x8000) & jnp.int32(-65536)
    r3 = r.reshape(DIM // 2, 2, BN)
    packed = lax.shift_right_logical(r3[:, 0, :], 16) | (r3[:, 1, :] & jnp.int32(-65536))
    xt4 = packed.T.reshape(4, BN // 4, DIM // 2)
    o_ref[...] = jnp.concatenate([xt4[0], xt4[1], xt4[2], xt4[3]], axis=1)


_pack_tc = pl.pallas_call(
    _pack_body,
    out_shape=jax.ShapeDtypeStruct((NBLK, PKW), jnp.int32),
    grid=(NSTEP,),
    in_specs=[pl.BlockSpec((DIM, BN), lambda j: (0, j))],
    out_specs=pl.BlockSpec((BN // 4, PKW), lambda j: (j, 0)),
)


def _sc_body(u_hbm, v_hbm, epk, cpk, out_hbm,
             idx_u, idx_v, blk_u, blk_v, ebuf, cbuf, out_v, sem_e, sem_c):
    wid = lax.axis_index("s") * NC + lax.axis_index("c")
    pltpu.sync_copy(u_hbm.at[wid], idx_u)
    pltpu.sync_copy(v_hbm.at[wid], idx_v)

    def blkids(g, carry):
        vu = idx_u[pl.ds(g * 16, 16)]
        vv = idx_v[pl.ds(g * 16, 16)]
        blk_u[pl.ds(g * 16, 16)] = ((vu >> 11) << 9) | (vu & 511)
        blk_v[pl.ds(g * 16, 16)] = ((vv >> 11) << 9) | (vv & 511)
        return carry

    lax.fori_loop(0, BPW // 16, blkids, 0)

    def fire(c, s):
        pltpu.async_copy(
            epk.at[blk_u.at[pl.ds(c * NI, NI)]], ebuf.at[s], sem_e.at[s])
        pltpu.async_copy(
            cpk.at[blk_v.at[pl.ds(c * NI, NI)]], cbuf.at[s], sem_c.at[s])

    def drain(s):
        pltpu.make_async_copy(
            epk.at[blk_u.at[pl.ds(0, NI)]], ebuf.at[s], sem_e.at[s]).wait()
        pltpu.make_async_copy(
            cpk.at[blk_v.at[pl.ds(0, NI)]], cbuf.at[s], sem_c.at[s]).wait()

    iota16 = lax.iota(jnp.int32, 16)
    perms = [iota16 ^ (1 << b) for b in range(4)]
    dnums = lax.GatherDimensionNumbers(
        offset_dims=(), collapsed_slice_dims=(0,), start_index_map=(0,))

    def permute(x, pm):
        return lax.gather(
            x, pm[:, None], dimension_numbers=dnums, slice_sizes=(1,),
            mode=lax.GatherScatterMode.PROMISE_IN_BOUNDS)

    def lanesum(p):
        for pm in perms:
            p = p + permute(p, pm)
        return p

    himask = jnp.full((16,), -65536, jnp.int32)

    def rowhalves(buf, k, off):
        w0 = buf[k, pl.ds(off, 16)]
        w1 = buf[k, pl.ds(off + 16, 16)]
        a0 = plsc.bitcast(w0 << 16, jnp.float32)
        b0 = plsc.bitcast(w0 & himask, jnp.float32)
        a1 = plsc.bitcast(w1 << 16, jnp.float32)
        b1 = plsc.bitcast(w1 & himask, jnp.float32)
        return a0, b0, a1, b1

    def compute(c, s):
        def body(g, carry):
            su = ((idx_u[pl.ds(c * NI + g * 16, 16)] >> 9) & 3) * 32
            sv = ((idx_v[pl.ds(c * NI + g * 16, 16)] >> 9) & 3) * 32
            acc = jnp.zeros((16,), jnp.float32)
            for k in range(16):
                ea0, eb0, ea1, eb1 = rowhalves(ebuf.at[s], g * 16 + k, su[k])
                ca0, cb0, ca1, cb1 = rowhalves(cbuf.at[s], g * 16 + k, sv[k])
                p = ea0 * ca0 + eb0 * cb0 + ea1 * ca1 + eb1 * cb1
                acc = jnp.where(iota16 == k, lanesum(p), acc)
            out_v[pl.ds(c * NI + g * 16, 16)] = acc
            return carry

        lax.fori_loop(0, NI // 16, body, 0)

    fire(0, 0)
    fire(1, 1)
    for c in range(NCHUNK):
        s = c % 2
        drain(s)
        compute(c, s)
        if c + 2 < NCHUNK:
            fire(c + 2, s)

    pltpu.sync_copy(out_v, out_hbm.at[pl.ds(wid * BPW, BPW)])


_sc_dot = pl.kernel(
    _sc_body,
    out_type=jax.ShapeDtypeStruct((BATCH,), jnp.float32),
    mesh=plsc.VectorSubcoreMesh(core_axis_name="c", subcore_axis_name="s"),
    compiler_params=pltpu.CompilerParams(needs_layout_passes=False),
    scratch_types=[
        pltpu.VMEM((BPW,), jnp.int32),
        pltpu.VMEM((BPW,), jnp.int32),
        pltpu.VMEM((BPW,), jnp.int32),
        pltpu.VMEM((BPW,), jnp.int32),
        pltpu.VMEM((2, NI, PKW), jnp.int32),
        pltpu.VMEM((2, NI, PKW), jnp.int32),
        pltpu.VMEM((BPW,), jnp.float32),
        pltpu.SemaphoreType.DMA((2,)),
        pltpu.SemaphoreType.DMA((2,)),
    ],
)


def _loss_body(p_ref, s_ref, o_ref):
    x = s_ref[...] * p_ref[...]
    ls = jnp.minimum(x, 0.0) - jnp.log1p(jnp.exp(-jnp.abs(x)))
    o_ref[0, 0] = -jnp.sum(ls)


_loss = pl.pallas_call(
    _loss_body,
    out_shape=jax.ShapeDtypeStruct((1, 1), jnp.float32),
    out_specs=pl.BlockSpec(memory_space=pltpu.SMEM),
)


def kernel(u, v, sign, emb_table, ctx_table):
    u2 = u.astype(jnp.int32).reshape(NW, BPW)
    v2 = v.astype(jnp.int32).reshape(NW, BPW)
    prod = _sc_dot(u2, v2, _pack_tc(emb_table.T), _pack_tc(ctx_table.T))
    loss = _loss(prod.reshape(128, 128), sign.reshape(128, 128))
    return loss.reshape(())


# speedup vs baseline: 62.1044x; 1.3307x over previous
import jax
import jax.numpy as jnp
from jax import lax
from jax.experimental import pallas as pl
from jax.experimental.pallas import tpu as pltpu
from jax.experimental.pallas import tpu_sc as plsc

NUM_NODES = 1000000
DIM = 64
BATCH = 16384

NC = 2
NS = 16
NW = NC * NS
BPW = BATCH // NW
CH = 128
NCH = BPW // CH
NSLOT = 2


def _sc_body(u_hbm, v_hbm, emb_hbm, ctx_hbm, out_hbm,
             idx_u, idx_v, erows, crows, out_v, sem_e, sem_c):
    wid = lax.axis_index("s") * NC + lax.axis_index("c")
    pltpu.sync_copy(u_hbm.at[wid], idx_u)
    pltpu.sync_copy(v_hbm.at[wid], idx_v)

    def fire(c, s):
        def body(g, carry):
            vu = idx_u[pl.ds(c * CH + g * 16, 16)]
            vv = idx_v[pl.ds(c * CH + g * 16, 16)]
            for k in range(16):
                r = g * 16 + k
                pltpu.async_copy(
                    emb_hbm.at[vu[k]], erows.at[s, r], sem_e.at[s])
                pltpu.async_copy(
                    ctx_hbm.at[vv[k]], crows.at[s, r], sem_c.at[s])
            return carry

        lax.fori_loop(0, CH // 16, body, 0)

    def drain(s):
        pltpu.make_async_copy(
            emb_hbm.at[pl.ds(0, CH)], erows.at[s], sem_e.at[s]).wait()
        pltpu.make_async_copy(
            ctx_hbm.at[pl.ds(0, CH)], crows.at[s], sem_c.at[s]).wait()

    iota16 = lax.iota(jnp.int32, 16)
    perms = [iota16 ^ (1 << b) for b in range(4)]
    dnums = lax.GatherDimensionNumbers(
        offset_dims=(), collapsed_slice_dims=(0,), start_index_map=(0,))

    def permute(x, pm):
        return lax.gather(
            x, pm[:, None], dimension_numbers=dnums, slice_sizes=(1,),
            mode=lax.GatherScatterMode.PROMISE_IN_BOUNDS)

    def lanesum(p):
        for pm in perms:
            p = p + permute(p, pm)
        return p

    def compute(c, s):
        def body(g, carry):
            acc = jnp.zeros((16,), jnp.float32)
            for k in range(16):
                r = g * 16 + k
                p = erows[s, r, pl.ds(0, 16)] * crows[s, r, pl.ds(0, 16)]
                p += erows[s, r, pl.ds(16, 16)] * crows[s, r, pl.ds(16, 16)]
                p += erows[s, r, pl.ds(32, 16)] * crows[s, r, pl.ds(32, 16)]
                p += erows[s, r, pl.ds(48, 16)] * crows[s, r, pl.ds(48, 16)]
                acc = jnp.where(iota16 == k, lanesum(p), acc)
            out_v[pl.ds(c * CH + g * 16, 16)] = acc
            return carry

        lax.fori_loop(0, CH // 16, body, 0)

    for c in range(NSLOT):
        fire(c, c)
    for c in range(NCH):
        s = c % NSLOT
        drain(s)
        compute(c, s)
        if c + NSLOT < NCH:
            fire(c + NSLOT, s)

    pltpu.sync_copy(out_v, out_hbm.at[pl.ds(wid * BPW, BPW)])


_sc_dot = pl.kernel(
    _sc_body,
    out_type=jax.ShapeDtypeStruct((BATCH,), jnp.float32),
    mesh=plsc.VectorSubcoreMesh(core_axis_name="c", subcore_axis_name="s"),
    scratch_types=[
        pltpu.VMEM((BPW,), jnp.int32),
        pltpu.VMEM((BPW,), jnp.int32),
        pltpu.VMEM((NSLOT, CH, DIM), jnp.float32),
        pltpu.VMEM((NSLOT, CH, DIM), jnp.float32),
        pltpu.VMEM((BPW,), jnp.float32),
        pltpu.SemaphoreType.DMA((NSLOT,)),
        pltpu.SemaphoreType.DMA((NSLOT,)),
    ],
)


def _loss_body(p_ref, s_ref, o_ref):
    x = s_ref[...] * p_ref[...]
    ls = jnp.minimum(x, 0.0) - jnp.log1p(jnp.exp(-jnp.abs(x)))
    o_ref[0, 0] = -jnp.sum(ls)


_loss = pl.pallas_call(
    _loss_body,
    out_shape=jax.ShapeDtypeStruct((1, 1), jnp.float32),
    out_specs=pl.BlockSpec(memory_space=pltpu.SMEM),
)


def kernel(u, v, sign, emb_table, ctx_table):
    u2 = u.astype(jnp.int32).reshape(NW, BPW)
    v2 = v.astype(jnp.int32).reshape(NW, BPW)
    prod = _sc_dot(u2, v2, emb_table, ctx_table)
    loss = _loss(prod.reshape(128, 128), sign.reshape(128, 128))
    return loss.reshape(())
